# baseline (speedup 1.0000x reference)
import functools

import jax
import jax.numpy as jnp
from jax import lax
from jax.experimental import pallas as pl
from jax.experimental.pallas import tpu as pltpu
from jax.experimental.pallas import tpu_sc as plsc

N_NODES = 10000
N_EDGES = 320000
D = 128
N_GRAPHS = 8

NPAD = 10240
N_CORES = 2
N_SUBCORES = 16
NW = N_CORES * N_SUBCORES
CHUNK = 128
CHUNKS_PER_W = 80
EPW = CHUNK * CHUNKS_PER_W
EPAD = NW * EPW
ROWS_PER_TILE = NPAD // N_SUBCORES

BLK = 1280


NBUF = 2
S_CH = 16
N_ST = CHUNKS_PER_W // S_CH


def _sc_segment_sum(feats, src3, dst3):
    mesh = plsc.VectorSubcoreMesh(core_axis_name="c", subcore_axis_name="s")

    @functools.partial(
        pl.kernel,
        out_type=jax.ShapeDtypeStruct((N_CORES * NPAD, D), jnp.float32),
        mesh=mesh,
        scratch_types=[
            pltpu.VMEM((S_CH, CHUNK), jnp.int32),
            pltpu.VMEM((S_CH, CHUNK), jnp.int32),
            [pltpu.VMEM((CHUNK, D), jnp.float32) for _ in range(NBUF)],
            [pltpu.SemaphoreType.DMA for _ in range(NBUF)],
            [pltpu.SemaphoreType.DMA for _ in range(NBUF)],
            pltpu.VMEM_SHARED((NPAD, D), jnp.float32),
        ],
    )
    def scatter_kernel(x_hbm, src_hbm, dst_hbm, out_hbm,
                       src_idx, dst_idx, rows, sem_g, sem_s, acc_sh):
        c = lax.axis_index("c")
        s = lax.axis_index("s")
        def zrow(i, _):
            for g in range(D // 16):
                rows[0][i, pl.ds(g * 16, 16)] = jnp.zeros((16,), jnp.float32)
            return ()

        lax.fori_loop(0, CHUNK, zrow, ())
        for t in range(ROWS_PER_TILE // CHUNK):
            pltpu.sync_copy(
                rows[0],
                acc_sh.at[pl.ds(s * ROWS_PER_TILE + t * CHUNK, CHUNK)])
        plsc.subcore_barrier()

        w = c * N_SUBCORES + s
        for st in range(N_ST):
            pltpu.sync_copy(src_hbm.at[w, pl.ds(st * S_CH, S_CH)], src_idx)
            pltpu.sync_copy(dst_hbm.at[w, pl.ds(st * S_CH, S_CH)], dst_idx)
            for b in range(NBUF):
                pltpu.async_copy(x_hbm.at[src_idx.at[b]], rows[b], sem_g[b])

            def body(k, _):
                for b in range(NBUF):
                    j = NBUF * k + b
                    pltpu.make_async_copy(
                        x_hbm.at[src_idx.at[j]], rows[b], sem_g[b]).wait()
                    pltpu.async_copy(
                        rows[b], acc_sh.at[dst_idx.at[j]], sem_s[b],
                        add=True)
                for b in range(NBUF):
                    j = NBUF * k + b
                    pltpu.make_async_copy(
                        rows[b], acc_sh.at[dst_idx.at[j]], sem_s[b]).wait()

                    @pl.when(j + NBUF < S_CH)
                    def _():
                        pltpu.async_copy(
                            x_hbm.at[src_idx.at[j + NBUF]], rows[b], sem_g[b])
                return ()

            lax.fori_loop(0, S_CH // NBUF, body, ())
        plsc.subcore_barrier()
        out_off = c * NPAD + s * ROWS_PER_TILE
        pltpu.sync_copy(acc_sh.at[pl.ds(s * ROWS_PER_TILE, ROWS_PER_TILE)],
                        out_hbm.at[pl.ds(out_off, ROWS_PER_TILE)])

    return scatter_kernel(feats, src3, dst3)


def _dense_relu(partials, x, W_rel, W_root, b):
    grid = NPAD // BLK

    def body(p_ref, x_ref, wr_ref, wk_ref, b_ref, o_ref):
        i = pl.program_id(0)
        agg = p_ref[0] + p_ref[1]
        h = jnp.dot(agg, wr_ref[...], preferred_element_type=jnp.float32)
        h = h + b_ref[...]
        rowid = i * BLK + lax.broadcasted_iota(jnp.int32, (BLK, 1), 0)
        h = h + jnp.dot(x_ref[...], wk_ref[...],
                        preferred_element_type=jnp.float32)
        h = jnp.maximum(h, 0.0)
        h = jnp.where(rowid < N_NODES, h, 0.0)
        o_ref[...] = h

    return pl.pallas_call(
        body,
        grid=(grid,),
        in_specs=[
            pl.BlockSpec((2, BLK, D), lambda i: (0, i, 0)),
            pl.BlockSpec((BLK, D), lambda i: (i, 0)),
            pl.BlockSpec((D, D), lambda i: (0, 0)),
            pl.BlockSpec((D, D), lambda i: (0, 0)),
            pl.BlockSpec((1, D), lambda i: (0, 0)),
        ],
        out_specs=pl.BlockSpec((BLK, D), lambda i: (i, 0)),
        out_shape=jax.ShapeDtypeStruct((NPAD, D), jnp.float32),
    )(partials, x, W_rel, W_root, b)


def _dense_pool(partials, h, W_rel, W_root, b, bounds):
    grid = NPAD // BLK

    def body(bounds_ref, p_ref, h_ref, wr_ref, wk_ref, b_ref, o_ref):
        i = pl.program_id(0)
        agg = p_ref[0] + p_ref[1]
        o = jnp.dot(agg, wr_ref[...], preferred_element_type=jnp.float32)
        o = o + b_ref[...]
        o = o + jnp.dot(h_ref[...], wk_ref[...],
                        preferred_element_type=jnp.float32)
        rid = i * BLK + lax.broadcasted_iota(jnp.int32, (BLK, 1), 0)

        @pl.when(i == 0)
        def _init():
            o_ref[...] = jnp.full((N_GRAPHS, D), -jnp.inf, jnp.float32)

        neg = jnp.float32(-jnp.inf)
        maxes = [
            jnp.max(
                jnp.where((rid >= bounds_ref[g]) & (rid < bounds_ref[g + 1]),
                          o, neg),
                axis=0)
            for g in range(N_GRAPHS)
        ]
        o_ref[...] = jnp.maximum(o_ref[...], jnp.stack(maxes, axis=0))

    grid_spec = pltpu.PrefetchScalarGridSpec(
        num_scalar_prefetch=1,
        grid=(grid,),
        in_specs=[
            pl.BlockSpec((2, BLK, D), lambda i, bd: (0, i, 0)),
            pl.BlockSpec((BLK, D), lambda i, bd: (i, 0)),
            pl.BlockSpec((D, D), lambda i, bd: (0, 0)),
            pl.BlockSpec((D, D), lambda i, bd: (0, 0)),
            pl.BlockSpec((1, D), lambda i, bd: (0, 0)),
        ],
        out_specs=pl.BlockSpec((N_GRAPHS, D), lambda i, bd: (0, 0)),
    )
    return pl.pallas_call(
        body,
        grid_spec=grid_spec,
        out_shape=jax.ShapeDtypeStruct((N_GRAPHS, D), jnp.float32),
    )(bounds, partials, h, W_rel, W_root, b)


def kernel(x, edge_index, batch, W_rel1, W_root1, b1, W_rel2, W_root2, b2):
    x = x.astype(jnp.float32)
    src = edge_index[0].astype(jnp.int32)
    dst = edge_index[1].astype(jnp.int32)
    batch = batch.astype(jnp.int32)

    pad_i = jnp.arange(EPAD - N_EDGES, dtype=jnp.int32)
    src_pad = (jnp.concatenate([src, pad_i % N_NODES])
               .reshape(NW, CHUNKS_PER_W, CHUNK))
    dst_pad = (jnp.concatenate([dst, N_NODES + pad_i % (NPAD - N_NODES)])
               .reshape(NW, CHUNKS_PER_W, CHUNK))
    bounds = jnp.searchsorted(
        batch, jnp.arange(N_GRAPHS + 1, dtype=jnp.int32)).astype(jnp.int32)
    b1r = b1.reshape(1, D)
    b2r = b2.reshape(1, D)

    p1 = _sc_segment_sum(x, src_pad, dst_pad)
    p1 = p1.reshape(N_CORES, NPAD, D)
    h = _dense_relu(p1, x, W_rel1, W_root1, b1r)

    p2 = _sc_segment_sum(h, src_pad, dst_pad)
    p2 = p2.reshape(N_CORES, NPAD, D)
    pooled = _dense_pool(p2, h, W_rel2, W_root2, b2r, bounds)
    return pooled


# speedup vs baseline: 1.3057x; 1.3057x over previous
import functools

import jax
import jax.numpy as jnp
from jax import lax
from jax.experimental import pallas as pl
from jax.experimental.pallas import tpu as pltpu
from jax.experimental.pallas import tpu_sc as plsc

N_NODES = 10000
N_EDGES = 320000
D = 128
N_GRAPHS = 8

NPAD = 10240
N_CORES = 2
N_SUBCORES = 16
NW = N_CORES * N_SUBCORES
CHUNK = 128
CHUNKS_PER_W = 80
EPW = CHUNK * CHUNKS_PER_W
EPAD = NW * EPW
ROWS_PER_TILE = NPAD // N_SUBCORES

BLK = 1280


NBUF = 2
S_CH = 40
N_ST = CHUNKS_PER_W // S_CH


def _sc_segment_sum(feats, src3, dst3):
    mesh = plsc.VectorSubcoreMesh(core_axis_name="c", subcore_axis_name="s")

    @functools.partial(
        pl.kernel,
        out_type=jax.ShapeDtypeStruct((N_CORES * NPAD, D), jnp.float32),
        mesh=mesh,
        scratch_types=[
            pltpu.VMEM((S_CH, CHUNK), jnp.int32),
            pltpu.VMEM((S_CH, CHUNK), jnp.int32),
            [pltpu.VMEM((CHUNK, D), jnp.float32) for _ in range(NBUF)],
            [pltpu.SemaphoreType.DMA for _ in range(NBUF)],
            [pltpu.SemaphoreType.DMA for _ in range(NBUF)],
            pltpu.VMEM_SHARED((NPAD, D), jnp.float32),
        ],
    )
    def scatter_kernel(x_hbm, src_hbm, dst_hbm, out_hbm,
                       src_idx, dst_idx, rows, sem_g, sem_s, acc_sh):
        c = lax.axis_index("c")
        s = lax.axis_index("s")
        def zrow(i, _):
            for g in range(D // 16):
                rows[0][i, pl.ds(g * 16, 16)] = jnp.zeros((16,), jnp.float32)
            return ()

        lax.fori_loop(0, CHUNK, zrow, ())
        for t in range(ROWS_PER_TILE // CHUNK):
            pltpu.sync_copy(
                rows[0],
                acc_sh.at[pl.ds(s * ROWS_PER_TILE + t * CHUNK, CHUNK)])
        plsc.subcore_barrier()

        w = c * N_SUBCORES + s
        for st in range(N_ST):
            pltpu.sync_copy(src_hbm.at[w, pl.ds(st * S_CH, S_CH)], src_idx)
            pltpu.sync_copy(dst_hbm.at[w, pl.ds(st * S_CH, S_CH)], dst_idx)
            for b in range(NBUF):
                pltpu.async_copy(x_hbm.at[src_idx.at[b]], rows[b], sem_g[b])

            def body(k, _):
                for b in range(NBUF):
                    j = NBUF * k + b
                    pltpu.make_async_copy(
                        x_hbm.at[src_idx.at[j]], rows[b], sem_g[b]).wait()
                    pltpu.async_copy(
                        rows[b], acc_sh.at[dst_idx.at[j]], sem_s[b],
                        add=True).wait()

                    @pl.when(j + NBUF < S_CH)
                    def _():
                        pltpu.async_copy(
                            x_hbm.at[src_idx.at[j + NBUF]], rows[b], sem_g[b])
                return ()

            lax.fori_loop(0, S_CH // NBUF, body, ())
        plsc.subcore_barrier()
        out_off = c * NPAD + s * ROWS_PER_TILE
        pltpu.sync_copy(acc_sh.at[pl.ds(s * ROWS_PER_TILE, ROWS_PER_TILE)],
                        out_hbm.at[pl.ds(out_off, ROWS_PER_TILE)])

    return scatter_kernel(feats, src3, dst3)


def _dense_relu(partials, x, W_rel, W_root, b):
    grid = NPAD // BLK

    def body(p_ref, x_ref, wr_ref, wk_ref, b_ref, o_ref):
        i = pl.program_id(0)
        agg = p_ref[0] + p_ref[1]
        h = jnp.dot(agg, wr_ref[...], preferred_element_type=jnp.float32)
        h = h + b_ref[...]
        rowid = i * BLK + lax.broadcasted_iota(jnp.int32, (BLK, 1), 0)
        h = h + jnp.dot(x_ref[...], wk_ref[...],
                        preferred_element_type=jnp.float32)
        h = jnp.maximum(h, 0.0)
        h = jnp.where(rowid < N_NODES, h, 0.0)
        o_ref[...] = h

    return pl.pallas_call(
        body,
        grid=(grid,),
        in_specs=[
            pl.BlockSpec((2, BLK, D), lambda i: (0, i, 0)),
            pl.BlockSpec((BLK, D), lambda i: (i, 0)),
            pl.BlockSpec((D, D), lambda i: (0, 0)),
            pl.BlockSpec((D, D), lambda i: (0, 0)),
            pl.BlockSpec((1, D), lambda i: (0, 0)),
        ],
        out_specs=pl.BlockSpec((BLK, D), lambda i: (i, 0)),
        out_shape=jax.ShapeDtypeStruct((NPAD, D), jnp.float32),
    )(partials, x, W_rel, W_root, b)


def _dense_pool(partials, h, W_rel, W_root, b, bounds):
    grid = NPAD // BLK

    def body(bounds_ref, p_ref, h_ref, wr_ref, wk_ref, b_ref, o_ref):
        i = pl.program_id(0)
        agg = p_ref[0] + p_ref[1]
        o = jnp.dot(agg, wr_ref[...], preferred_element_type=jnp.float32)
        o = o + b_ref[...]
        o = o + jnp.dot(h_ref[...], wk_ref[...],
                        preferred_element_type=jnp.float32)
        rid = i * BLK + lax.broadcasted_iota(jnp.int32, (BLK, 1), 0)

        @pl.when(i == 0)
        def _init():
            o_ref[...] = jnp.full((N_GRAPHS, D), -jnp.inf, jnp.float32)

        neg = jnp.float32(-jnp.inf)
        maxes = [
            jnp.max(
                jnp.where((rid >= bounds_ref[g]) & (rid < bounds_ref[g + 1]),
                          o, neg),
                axis=0)
            for g in range(N_GRAPHS)
        ]
        o_ref[...] = jnp.maximum(o_ref[...], jnp.stack(maxes, axis=0))

    grid_spec = pltpu.PrefetchScalarGridSpec(
        num_scalar_prefetch=1,
        grid=(grid,),
        in_specs=[
            pl.BlockSpec((2, BLK, D), lambda i, bd: (0, i, 0)),
            pl.BlockSpec((BLK, D), lambda i, bd: (i, 0)),
            pl.BlockSpec((D, D), lambda i, bd: (0, 0)),
            pl.BlockSpec((D, D), lambda i, bd: (0, 0)),
            pl.BlockSpec((1, D), lambda i, bd: (0, 0)),
        ],
        out_specs=pl.BlockSpec((N_GRAPHS, D), lambda i, bd: (0, 0)),
    )
    return pl.pallas_call(
        body,
        grid_spec=grid_spec,
        out_shape=jax.ShapeDtypeStruct((N_GRAPHS, D), jnp.float32),
    )(bounds, partials, h, W_rel, W_root, b)


def kernel(x, edge_index, batch, W_rel1, W_root1, b1, W_rel2, W_root2, b2):
    x = x.astype(jnp.float32)
    src = edge_index[0].astype(jnp.int32)
    dst = edge_index[1].astype(jnp.int32)
    batch = batch.astype(jnp.int32)

    pad_i = jnp.arange(EPAD - N_EDGES, dtype=jnp.int32)
    src_pad = (jnp.concatenate([src, pad_i % N_NODES])
               .reshape(NW, CHUNKS_PER_W, CHUNK))
    dst_pad = (jnp.concatenate([dst, N_NODES + pad_i % (NPAD - N_NODES)])
               .reshape(NW, CHUNKS_PER_W, CHUNK))
    bounds = jnp.searchsorted(
        batch, jnp.arange(N_GRAPHS + 1, dtype=jnp.int32)).astype(jnp.int32)
    b1r = b1.reshape(1, D)
    b2r = b2.reshape(1, D)

    p1 = _sc_segment_sum(x, src_pad, dst_pad)
    p1 = p1.reshape(N_CORES, NPAD, D)
    h = _dense_relu(p1, x, W_rel1, W_root1, b1r)

    p2 = _sc_segment_sum(h, src_pad, dst_pad)
    p2 = p2.reshape(N_CORES, NPAD, D)
    pooled = _dense_pool(p2, h, W_rel2, W_root2, b2r, bounds)
    return pooled


# speedup vs baseline: 1.3086x; 1.0022x over previous
import functools

import jax
import jax.numpy as jnp
from jax import lax
from jax.experimental import pallas as pl
from jax.experimental.pallas import tpu as pltpu
from jax.experimental.pallas import tpu_sc as plsc

N_NODES = 10000
N_EDGES = 320000
D = 128
N_GRAPHS = 8

NPAD = 10240
N_CORES = 2
N_SUBCORES = 16
NW = N_CORES * N_SUBCORES
CHUNK = 128
CHUNKS_PER_W = 80
EPW = CHUNK * CHUNKS_PER_W
EPAD = NW * EPW
ROWS_PER_TILE = NPAD // N_SUBCORES

BLK = 1280


NBUF = 2
S_CH = 40
N_ST = CHUNKS_PER_W // S_CH


def _sc_segment_sum(feats, edges4):
    mesh = plsc.VectorSubcoreMesh(core_axis_name="c", subcore_axis_name="s")

    @functools.partial(
        pl.kernel,
        out_type=jax.ShapeDtypeStruct((N_CORES * NPAD, D), jnp.float32),
        mesh=mesh,
        scratch_types=[
            pltpu.VMEM((2, S_CH, CHUNK), jnp.int32),
            [pltpu.VMEM((CHUNK, D), jnp.float32) for _ in range(NBUF)],
            [pltpu.SemaphoreType.DMA for _ in range(NBUF)],
            [pltpu.SemaphoreType.DMA for _ in range(NBUF)],
            pltpu.VMEM_SHARED((NPAD, D), jnp.float32),
        ],
    )
    def scatter_kernel(x_hbm, edge_hbm, out_hbm,
                       eidx, rows, sem_g, sem_s, acc_sh):
        c = lax.axis_index("c")
        s = lax.axis_index("s")
        def zrow(i, _):
            for g in range(D // 16):
                rows[0][i, pl.ds(g * 16, 16)] = jnp.zeros((16,), jnp.float32)
            return ()

        lax.fori_loop(0, CHUNK, zrow, ())
        for t in range(ROWS_PER_TILE // CHUNK):
            pltpu.sync_copy(
                rows[0],
                acc_sh.at[pl.ds(s * ROWS_PER_TILE + t * CHUNK, CHUNK)])
        plsc.subcore_barrier()

        w = c * N_SUBCORES + s
        for st in range(N_ST):
            pltpu.sync_copy(edge_hbm.at[w, :, pl.ds(st * S_CH, S_CH)], eidx)
            for b in range(NBUF):
                pltpu.async_copy(x_hbm.at[eidx.at[0, b]], rows[b], sem_g[b])

            def body(k, _):
                for b in range(NBUF):
                    j = NBUF * k + b
                    pltpu.make_async_copy(
                        x_hbm.at[eidx.at[0, j]], rows[b], sem_g[b]).wait()
                    pltpu.async_copy(
                        rows[b], acc_sh.at[eidx.at[1, j]], sem_s[b],
                        add=True).wait()

                    @pl.when(j + NBUF < S_CH)
                    def _():
                        pltpu.async_copy(
                            x_hbm.at[eidx.at[0, j + NBUF]], rows[b], sem_g[b])
                return ()

            lax.fori_loop(0, S_CH // NBUF, body, ())
        plsc.subcore_barrier()
        out_off = c * NPAD + s * ROWS_PER_TILE
        pltpu.sync_copy(acc_sh.at[pl.ds(s * ROWS_PER_TILE, ROWS_PER_TILE)],
                        out_hbm.at[pl.ds(out_off, ROWS_PER_TILE)])

    return scatter_kernel(feats, edges4)


def _dense_relu(partials, x, W_rel, W_root, b):
    grid = NPAD // BLK

    def body(p_ref, x_ref, wr_ref, wk_ref, b_ref, o_ref):
        i = pl.program_id(0)
        agg = p_ref[0] + p_ref[1]
        h = jnp.dot(agg, wr_ref[...], preferred_element_type=jnp.float32)
        h = h + b_ref[...]
        rowid = i * BLK + lax.broadcasted_iota(jnp.int32, (BLK, 1), 0)
        h = h + jnp.dot(x_ref[...], wk_ref[...],
                        preferred_element_type=jnp.float32)
        h = jnp.maximum(h, 0.0)
        h = jnp.where(rowid < N_NODES, h, 0.0)
        o_ref[...] = h

    return pl.pallas_call(
        body,
        grid=(grid,),
        in_specs=[
            pl.BlockSpec((2, BLK, D), lambda i: (0, i, 0)),
            pl.BlockSpec((BLK, D), lambda i: (i, 0)),
            pl.BlockSpec((D, D), lambda i: (0, 0)),
            pl.BlockSpec((D, D), lambda i: (0, 0)),
            pl.BlockSpec((1, D), lambda i: (0, 0)),
        ],
        out_specs=pl.BlockSpec((BLK, D), lambda i: (i, 0)),
        out_shape=jax.ShapeDtypeStruct((NPAD, D), jnp.float32),
    )(partials, x, W_rel, W_root, b)


def _dense_pool(partials, h, W_rel, W_root, b, bounds):
    grid = NPAD // BLK

    def body(bounds_ref, p_ref, h_ref, wr_ref, wk_ref, b_ref, o_ref):
        i = pl.program_id(0)
        agg = p_ref[0] + p_ref[1]
        o = jnp.dot(agg, wr_ref[...], preferred_element_type=jnp.float32)
        o = o + b_ref[...]
        o = o + jnp.dot(h_ref[...], wk_ref[...],
                        preferred_element_type=jnp.float32)
        rid = i * BLK + lax.broadcasted_iota(jnp.int32, (BLK, 1), 0)

        @pl.when(i == 0)
        def _init():
            o_ref[...] = jnp.full((N_GRAPHS, D), -jnp.inf, jnp.float32)

        neg = jnp.float32(-jnp.inf)
        maxes = [
            jnp.max(
                jnp.where((rid >= bounds_ref[g]) & (rid < bounds_ref[g + 1]),
                          o, neg),
                axis=0)
            for g in range(N_GRAPHS)
        ]
        o_ref[...] = jnp.maximum(o_ref[...], jnp.stack(maxes, axis=0))

    grid_spec = pltpu.PrefetchScalarGridSpec(
        num_scalar_prefetch=1,
        grid=(grid,),
        in_specs=[
            pl.BlockSpec((2, BLK, D), lambda i, bd: (0, i, 0)),
            pl.BlockSpec((BLK, D), lambda i, bd: (i, 0)),
            pl.BlockSpec((D, D), lambda i, bd: (0, 0)),
            pl.BlockSpec((D, D), lambda i, bd: (0, 0)),
            pl.BlockSpec((1, D), lambda i, bd: (0, 0)),
        ],
        out_specs=pl.BlockSpec((N_GRAPHS, D), lambda i, bd: (0, 0)),
    )
    return pl.pallas_call(
        body,
        grid_spec=grid_spec,
        out_shape=jax.ShapeDtypeStruct((N_GRAPHS, D), jnp.float32),
    )(bounds, partials, h, W_rel, W_root, b)


def kernel(x, edge_index, batch, W_rel1, W_root1, b1, W_rel2, W_root2, b2):
    x = x.astype(jnp.float32)
    src = edge_index[0].astype(jnp.int32)
    dst = edge_index[1].astype(jnp.int32)
    batch = batch.astype(jnp.int32)

    pad_i = jnp.arange(EPAD - N_EDGES, dtype=jnp.int32)
    src_pad = (jnp.concatenate([src, pad_i % N_NODES])
               .reshape(NW, 1, CHUNKS_PER_W, CHUNK))
    dst_pad = (jnp.concatenate([dst, N_NODES + pad_i % (NPAD - N_NODES)])
               .reshape(NW, 1, CHUNKS_PER_W, CHUNK))
    edges4 = jnp.concatenate([src_pad, dst_pad], axis=1)
    bounds = jnp.searchsorted(
        batch, jnp.arange(N_GRAPHS + 1, dtype=jnp.int32)).astype(jnp.int32)
    b1r = b1.reshape(1, D)
    b2r = b2.reshape(1, D)

    p1 = _sc_segment_sum(x, edges4)
    p1 = p1.reshape(N_CORES, NPAD, D)
    h = _dense_relu(p1, x, W_rel1, W_root1, b1r)

    p2 = _sc_segment_sum(h, edges4)
    p2 = p2.reshape(N_CORES, NPAD, D)
    pooled = _dense_pool(p2, h, W_rel2, W_root2, b2r, bounds)
    return pooled
